# baseline (speedup 1.0000x reference)
import functools

import jax
import jax.numpy as jnp
from jax import lax
from jax.experimental import pallas as pl
from jax.experimental.pallas import tpu as pltpu
from jax.experimental.pallas import tpu_sc as plsc

BATCH = 16384
EMBED = 32
NC = 2
NS = 16
NW = NC * NS
ROWS_PER_W = BATCH // NW
CHUNK = 128
NCHUNK = ROWS_PER_W // CHUNK
NGROUP = ROWS_PER_W // 16


@functools.partial(
    pl.kernel,
    out_type=jax.ShapeDtypeStruct((BATCH,), jnp.float32),
    mesh=plsc.VectorSubcoreMesh(core_axis_name="c", subcore_axis_name="s"),
    scratch_types=[
        pltpu.VMEM((NCHUNK, CHUNK), jnp.int32),
        pltpu.VMEM((NCHUNK, CHUNK), jnp.int32),
        pltpu.VMEM((2 * ROWS_PER_W, EMBED), jnp.float32),
        pltpu.VMEM((2 * EMBED + 1, 16), jnp.float32),
        pltpu.VMEM((ROWS_PER_W,), jnp.float32),
        pltpu.SemaphoreType.DMA,
    ],
    compiler_params=pltpu.CompilerParams(
        needs_layout_passes=False, use_tc_tiling_on_sc=False),
)
def _sc_rating_kernel(uids, aids, utab, atab, wtile, out,
                      uidx_v, aidx_v, rows_v, wb_v, out_v, sem):
    wid = lax.axis_index("s") * NC + lax.axis_index("c")
    base = wid * ROWS_PER_W

    pltpu.sync_copy(uids.at[wid], uidx_v)
    pltpu.sync_copy(aids.at[wid], aidx_v)
    pltpu.sync_copy(wtile, wb_v)

    copies = []
    for j in range(NCHUNK):
        copies.append(pltpu.async_copy(
            utab.at[uidx_v.at[j]],
            rows_v.at[pl.ds(j * CHUNK, CHUNK)], sem))
        copies.append(pltpu.async_copy(
            atab.at[aidx_v.at[j]],
            rows_v.at[pl.ds(ROWS_PER_W + j * CHUNK, CHUNK)], sem))
    for c in copies:
        c.wait()

    bias_vec = wb_v[2 * EMBED, :]
    for g in range(NGROUP):
        out_v[pl.ds(g * 16, 16)] = bias_vec

    iota16 = jnp.arange(16, dtype=jnp.int32)

    def dim_step(d, carry):
        wu = wb_v[d, :]
        wa = wb_v[d + EMBED, :]
        col = jnp.full((16,), d, dtype=jnp.int32)
        for g in range(NGROUP):
            row = iota16 + (g * 16)
            colu = plsc.load_gather(rows_v, [row, col])
            cola = plsc.load_gather(rows_v, [row + ROWS_PER_W, col])
            plsc.addupdate(out_v.at[pl.ds(g * 16, 16)], colu * wu + cola * wa)
        return carry

    lax.fori_loop(0, EMBED, dim_step, 0)

    pltpu.sync_copy(out_v, out.at[pl.ds(base, ROWS_PER_W)])


def kernel(user_ids, article_ids, user_table, article_table, fc_w, fc_b):
    uids = user_ids.astype(jnp.int32).reshape(NW, NCHUNK, CHUNK)
    aids = article_ids.astype(jnp.int32).reshape(NW, NCHUNK, CHUNK)
    wb = jnp.concatenate([fc_w.reshape(2 * EMBED), fc_b]).astype(jnp.float32)
    wtile = jnp.broadcast_to(wb[:, None], (2 * EMBED + 1, 16))
    out = _sc_rating_kernel(uids, aids, user_table, article_table, wtile)
    return out.reshape(BATCH, 1)


# speedup vs baseline: 5.3981x; 5.3981x over previous
import functools

import jax
import jax.numpy as jnp
from jax import lax
from jax.experimental import pallas as pl
from jax.experimental.pallas import tpu as pltpu
from jax.experimental.pallas import tpu_sc as plsc

BATCH = 16384
EMBED = 32
NROWS = 1000000
NC = 2
NS = 16
NW = NC * NS
ROWS_PER_W = BATCH // NW
CHUNK = 128
NCHUNK = ROWS_PER_W // CHUNK

BLK = 32768
NBLK = -(-NROWS // BLK)
SPAD = NBLK * BLK


def _tc_score_body(w_ref, u_ref, a_ref, su_ref, sa_ref):
    wu = w_ref[:, :EMBED]
    wa = w_ref[:, EMBED:]
    dn = (((1,), (0,)), ((), ()))
    su_ref[...] = lax.dot_general(
        wu, u_ref[...], dn, precision=lax.Precision.HIGHEST,
        preferred_element_type=jnp.float32)
    sa_ref[...] = lax.dot_general(
        wa, a_ref[...], dn, precision=lax.Precision.HIGHEST,
        preferred_element_type=jnp.float32)


_tc_score = pl.pallas_call(
    _tc_score_body,
    grid=(NBLK,),
    in_specs=[
        pl.BlockSpec((1, 2 * EMBED), lambda i: (0, 0)),
        pl.BlockSpec((EMBED, BLK), lambda i: (0, i)),
        pl.BlockSpec((EMBED, BLK), lambda i: (0, i)),
    ],
    out_specs=[
        pl.BlockSpec((1, BLK), lambda i: (0, i)),
        pl.BlockSpec((1, BLK), lambda i: (0, i)),
    ],
    out_shape=[
        jax.ShapeDtypeStruct((1, SPAD), jnp.float32),
        jax.ShapeDtypeStruct((1, SPAD), jnp.float32),
    ],
    compiler_params=pltpu.CompilerParams(
        dimension_semantics=("arbitrary",)),
)


@functools.partial(
    pl.kernel,
    out_type=jax.ShapeDtypeStruct((BATCH,), jnp.float32),
    mesh=plsc.VectorSubcoreMesh(core_axis_name="c", subcore_axis_name="s"),
    scratch_types=[
        pltpu.VMEM((NCHUNK, CHUNK), jnp.int32),
        pltpu.VMEM((NCHUNK, CHUNK), jnp.int32),
        pltpu.VMEM((ROWS_PER_W,), jnp.float32),
        pltpu.VMEM((ROWS_PER_W,), jnp.float32),
        pltpu.VMEM((16,), jnp.float32),
        pltpu.VMEM((ROWS_PER_W,), jnp.float32),
        pltpu.SemaphoreType.DMA,
    ],
    compiler_params=pltpu.CompilerParams(
        needs_layout_passes=False, use_tc_tiling_on_sc=False),
)
def _sc_combine_kernel(uids, aids, su, sa, b16, out,
                       uidx_v, aidx_v, su_v, sa_v, b_v, out_v, sem):
    wid = lax.axis_index("s") * NC + lax.axis_index("c")
    base = wid * ROWS_PER_W

    pltpu.sync_copy(uids.at[wid], uidx_v)
    pltpu.sync_copy(aids.at[wid], aidx_v)
    pltpu.sync_copy(b16, b_v)

    copies = []
    for j in range(NCHUNK):
        copies.append(pltpu.async_copy(
            su.at[uidx_v.at[j]], su_v.at[pl.ds(j * CHUNK, CHUNK)], sem))
        copies.append(pltpu.async_copy(
            sa.at[aidx_v.at[j]], sa_v.at[pl.ds(j * CHUNK, CHUNK)], sem))
    for c in copies:
        c.wait()

    bias_vec = b_v[...]
    for g in range(ROWS_PER_W // 16):
        sl = pl.ds(g * 16, 16)
        out_v[sl] = su_v[sl] + sa_v[sl] + bias_vec

    pltpu.sync_copy(out_v, out.at[pl.ds(base, ROWS_PER_W)])


def kernel(user_ids, article_ids, user_table, article_table, fc_w, fc_b):
    uids = user_ids.astype(jnp.int32).reshape(NW, NCHUNK, CHUNK)
    aids = article_ids.astype(jnp.int32).reshape(NW, NCHUNK, CHUNK)
    su2, sa2 = _tc_score(fc_w, user_table.T, article_table.T)
    su = su2.reshape(SPAD)
    sa = sa2.reshape(SPAD)
    b16 = jnp.broadcast_to(fc_b.astype(jnp.float32), (16,))
    out = _sc_combine_kernel(uids, aids, su, sa, b16)
    return out.reshape(BATCH, 1)


# speedup vs baseline: 8.6898x; 1.6098x over previous
import functools

import jax
import jax.numpy as jnp
from jax import lax
from jax.experimental import pallas as pl
from jax.experimental.pallas import tpu as pltpu
from jax.experimental.pallas import tpu_sc as plsc

BATCH = 16384
EMBED = 32
NROWS = 1000000
NC = 2
NS = 16
NW = NC * NS
ROWS_PER_W = BATCH // NW
CHUNK = 128
NCHUNK = ROWS_PER_W // CHUNK

BLK = 32768
NBLK = -(-NROWS // BLK)
SPAD = NBLK * BLK


def _tc_score_body(w_ref, u_ref, a_ref, su_ref, sa_ref):
    su_ref[...] = jnp.sum(u_ref[...] * w_ref[:, 0:1], axis=0, keepdims=True)
    sa_ref[...] = jnp.sum(a_ref[...] * w_ref[:, 1:2], axis=0, keepdims=True)


_tc_score = pl.pallas_call(
    _tc_score_body,
    grid=(NBLK,),
    in_specs=[
        pl.BlockSpec((EMBED, 2), lambda i: (0, 0)),
        pl.BlockSpec((EMBED, BLK), lambda i: (0, i)),
        pl.BlockSpec((EMBED, BLK), lambda i: (0, i)),
    ],
    out_specs=[
        pl.BlockSpec((1, BLK), lambda i: (0, i)),
        pl.BlockSpec((1, BLK), lambda i: (0, i)),
    ],
    out_shape=[
        jax.ShapeDtypeStruct((1, SPAD), jnp.float32),
        jax.ShapeDtypeStruct((1, SPAD), jnp.float32),
    ],
    compiler_params=pltpu.CompilerParams(
        dimension_semantics=("arbitrary",)),
)


@functools.partial(
    pl.kernel,
    out_type=jax.ShapeDtypeStruct((BATCH,), jnp.float32),
    mesh=plsc.VectorSubcoreMesh(core_axis_name="c", subcore_axis_name="s"),
    scratch_types=[
        pltpu.VMEM((NCHUNK, CHUNK), jnp.int32),
        pltpu.VMEM((NCHUNK, CHUNK), jnp.int32),
        pltpu.VMEM((ROWS_PER_W,), jnp.float32),
        pltpu.VMEM((ROWS_PER_W,), jnp.float32),
        pltpu.VMEM((16,), jnp.float32),
        pltpu.VMEM((ROWS_PER_W,), jnp.float32),
        pltpu.SemaphoreType.DMA,
    ],
    compiler_params=pltpu.CompilerParams(
        needs_layout_passes=False, use_tc_tiling_on_sc=False),
)
def _sc_combine_kernel(uids, aids, su, sa, b16, out,
                       uidx_v, aidx_v, su_v, sa_v, b_v, out_v, sem):
    wid = lax.axis_index("s") * NC + lax.axis_index("c")
    base = wid * ROWS_PER_W

    pltpu.sync_copy(uids.at[wid], uidx_v)
    pltpu.sync_copy(aids.at[wid], aidx_v)
    pltpu.sync_copy(b16, b_v)

    copies = []
    for j in range(NCHUNK):
        copies.append(pltpu.async_copy(
            su.at[uidx_v.at[j]], su_v.at[pl.ds(j * CHUNK, CHUNK)], sem))
        copies.append(pltpu.async_copy(
            sa.at[aidx_v.at[j]], sa_v.at[pl.ds(j * CHUNK, CHUNK)], sem))
    for c in copies:
        c.wait()

    bias_vec = b_v[...]
    for g in range(ROWS_PER_W // 16):
        sl = pl.ds(g * 16, 16)
        out_v[sl] = su_v[sl] + sa_v[sl] + bias_vec

    pltpu.sync_copy(out_v, out.at[pl.ds(base, ROWS_PER_W)])


def kernel(user_ids, article_ids, user_table, article_table, fc_w, fc_b):
    uids = user_ids.astype(jnp.int32).reshape(NW, NCHUNK, CHUNK)
    aids = article_ids.astype(jnp.int32).reshape(NW, NCHUNK, CHUNK)
    wcols = fc_w.reshape(2, EMBED).T
    su2, sa2 = _tc_score(wcols, user_table.T, article_table.T)
    su = su2.reshape(SPAD)
    sa = sa2.reshape(SPAD)
    b16 = jnp.broadcast_to(fc_b.astype(jnp.float32), (16,))
    out = _sc_combine_kernel(uids, aids, su, sa, b16)
    return out.reshape(BATCH, 1)
